# baseline (speedup 1.0000x reference)
import functools

import jax
import jax.numpy as jnp
from jax import lax
from jax.experimental import pallas as pl
from jax.experimental.pallas import tpu as pltpu
from jax.experimental.pallas import tpu_sc as plsc

_STEPLIM = 5
_TOL = 1e-4
_NB = 512


def _knn_body(a_ref, p2t_ref, dist_ref, idx_ref):
    b = pl.program_id(0)
    nb_total = pl.num_programs(0)
    a = a_ref[0]
    p2t = p2t_ref[0]

    m = p2t.shape[1]
    a2 = jnp.sum(a * a, axis=1, keepdims=True)
    b2 = (p2t[0:1, :] * p2t[0:1, :] + p2t[1:2, :] * p2t[1:2, :]
          + p2t[2:3, :] * p2t[2:3, :])
    cross = lax.dot_general(a.astype(jnp.bfloat16), p2t.astype(jnp.bfloat16),
                            (((1,), (0,)), ((), ())),
                            preferred_element_type=jnp.float32)
    d2 = a2 + b2 - 2.0 * cross
    rowmin = jnp.maximum(jnp.min(d2, axis=1, keepdims=True), 0.0)
    dist_ref[...] = jnp.sqrt(rowmin)[None]

    @pl.when(b == nb_total - 1)
    def _():
        iota = lax.broadcasted_iota(jnp.int32, d2.shape, 1)
        cand = jnp.where(d2 <= rowmin, iota, m)
        idx_ref[...] = jnp.min(cand, axis=1, keepdims=True)[None]


def _make_knn_call(B, N, M, nb):
    nblk = N // nb
    return pl.pallas_call(
        _knn_body,
        grid=(B, nblk),
        in_specs=[
            pl.BlockSpec((1, nb, 3), lambda b, j: (b, j, 0)),
            pl.BlockSpec((1, 3, M), lambda b, j: (b, 0, 0)),
        ],
        out_specs=[
            pl.BlockSpec((1, nb, 1), lambda b, j: (b * nblk + j, 0, 0)),
            pl.BlockSpec((1, nb, 1), lambda b, j: (j, 0, 0)),
        ],
        out_shape=[
            jax.ShapeDtypeStruct((B * nblk, nb, 1), jnp.float32),
            jax.ShapeDtypeStruct((nblk, nb, 1), jnp.int32),
        ],
        compiler_params=pltpu.CompilerParams(
            dimension_semantics=("arbitrary", "arbitrary")),
    )


def _make_sc_gather(V, D, BB):
    info = plsc.get_sparse_core_info()
    nc, ns = info.num_cores, info.num_subcores
    nw = nc * ns
    b_per_w = BB // nw
    mesh = plsc.VectorSubcoreMesh(core_axis_name="c", subcore_axis_name="s")

    @functools.partial(
        pl.kernel, mesh=mesh,
        out_type=jax.ShapeDtypeStruct((BB, D), jnp.float32),
        scratch_types=[
            pltpu.VMEM((b_per_w,), jnp.int32),
            pltpu.VMEM((b_per_w, D), jnp.float32),
            pltpu.SemaphoreType.DMA,
        ],
    )
    def gather_k(table_hbm, idx_hbm, out_hbm, idx_v, rows_v, sem):
        wid = lax.axis_index("s") * nc + lax.axis_index("c")
        base = wid * b_per_w
        pltpu.sync_copy(idx_hbm.at[pl.ds(base, b_per_w)], idx_v)
        pltpu.async_copy(table_hbm.at[idx_v], rows_v, sem).wait()
        pltpu.sync_copy(rows_v, out_hbm.at[pl.ds(base, b_per_w)])

    return gather_k


def _quat_rot(H, sweeps=7):
    Sxx, Sxy, Sxz = H[..., 0, 0], H[..., 0, 1], H[..., 0, 2]
    Syx, Syy, Syz = H[..., 1, 0], H[..., 1, 1], H[..., 1, 2]
    Szx, Szy, Szz = H[..., 2, 0], H[..., 2, 1], H[..., 2, 2]
    A = jnp.stack([
        jnp.stack([Sxx + Syy + Szz, Syz - Szy, Szx - Sxz, Sxy - Syx], -1),
        jnp.stack([Syz - Szy, Sxx - Syy - Szz, Sxy + Syx, Szx + Sxz], -1),
        jnp.stack([Szx - Sxz, Sxy + Syx, -Sxx + Syy - Szz, Syz + Szy], -1),
        jnp.stack([Sxy - Syx, Szx + Sxz, Syz + Szy, -Sxx - Syy + Szz], -1),
    ], -2)
    Q = jnp.broadcast_to(jnp.eye(4, dtype=H.dtype), A.shape)
    for _ in range(sweeps):
        for (p, q) in ((0, 1), (0, 2), (0, 3), (1, 2), (1, 3), (2, 3)):
            x = A[..., p, p] - A[..., q, q]
            y = 2.0 * A[..., p, q]
            h = jnp.sqrt(x * x + y * y)
            ynz = y != 0.0
            t = jnp.where(x >= 0.0,
                          y / jnp.where(h + x > 0.0, h + x, 1.0),
                          jnp.where(ynz, (h - x) / jnp.where(ynz, y, 1.0),
                                    0.0))
            c = (1.0 / jnp.sqrt(1.0 + t * t))[..., None]
            s = (t[..., None]) * c
            Ap, Aq = A[..., p, :], A[..., q, :]
            A = A.at[..., p, :].set(c * Ap + s * Aq)
            A = A.at[..., q, :].set(-s * Ap + c * Aq)
            Ap, Aq = A[..., :, p], A[..., :, q]
            A = A.at[..., :, p].set(c * Ap + s * Aq)
            A = A.at[..., :, q].set(-s * Ap + c * Aq)
            Qp, Qq = Q[..., :, p], Q[..., :, q]
            Q = Q.at[..., :, p].set(c * Qp + s * Qq)
            Q = Q.at[..., :, q].set(-s * Qp + c * Qq)
    evals = jnp.diagonal(A, axis1=-2, axis2=-1)
    k = jnp.argmax(evals, axis=-1)
    qv = jnp.take_along_axis(
        Q, jnp.broadcast_to(k[..., None, None], Q.shape[:-1] + (1,)), axis=-1
    )[..., 0]
    qv = qv / jnp.sqrt(jnp.sum(qv * qv, axis=-1, keepdims=True))
    w, x, y, z = qv[..., 0], qv[..., 1], qv[..., 2], qv[..., 3]
    return jnp.stack([
        jnp.stack([1 - 2 * (y * y + z * z), 2 * (x * y - w * z),
                   2 * (x * z + w * y)], -1),
        jnp.stack([2 * (x * y + w * z), 1 - 2 * (x * x + z * z),
                   2 * (y * z - w * x)], -1),
        jnp.stack([2 * (x * z - w * y), 2 * (y * z + w * x),
                   1 - 2 * (x * x + y * y)], -1),
    ], -2)


def _ptransform_svd(pa, pb):
    c1 = jnp.mean(pa, axis=-2, keepdims=True)
    c2 = jnp.mean(pb, axis=-2, keepdims=True)
    H = jnp.einsum('bni,bnj->bij', pa - c1, pb - c2)
    U, S, Vt = jnp.linalg.svd(H)
    V = jnp.swapaxes(Vt, -1, -2)
    Ut = jnp.swapaxes(U, -1, -2)
    d = jnp.linalg.det(jnp.matmul(V, Ut))
    s = jnp.where(d < 0, -1.0, 1.0)
    D = jnp.stack([jnp.ones_like(s), jnp.ones_like(s), s], axis=-1)
    R = jnp.matmul(V * D[..., None, :], Ut)
    t = c2[..., 0, :] - jnp.einsum('bij,bj->bi', R, c1[..., 0, :])
    return R, t


def _ptransform_quat(pa, pb):
    c1 = jnp.mean(pa, axis=-2, keepdims=True)
    c2 = jnp.mean(pb, axis=-2, keepdims=True)
    H = jnp.einsum('bni,bnj->bij', pa - c1, pb - c2)
    R = _quat_rot(H)
    t = c2[..., 0, :] - jnp.einsum('bij,bj->bi', R, c1[..., 0, :])
    return R, t


def kernel(p1, p2):
    B, N, _ = p1.shape
    M = p2.shape[1]
    p2t = jnp.swapaxes(p2, 1, 2)
    table = jnp.pad(p2, ((0, 0), (0, 0), (0, 125))).reshape(B * M, 128)
    offs = (jnp.arange(B, dtype=jnp.int32) * M)[:, None]

    knn_call = _make_knn_call(B, N, M, _NB)
    sc_gather = _make_sc_gather(B * M, 128, B * N)
    nblk = N // _NB

    def cond_fn(carry):
        it, temppc, err, have_err, done = carry
        return (it <= _STEPLIM) & jnp.logical_not(done)

    def body_fn(carry):
        it, temppc, err, have_err, done = carry
        it = it + 1
        dist_o, idx_o = knn_call(temppc, p2t)
        knndist = dist_o.reshape(B, N)
        idx_last = idx_o.reshape(N)
        errnew = jnp.mean(knndist, axis=-1)
        converged = have_err & jnp.all(jnp.abs((errnew - err) / err) < _TOL)
        idx_all = (idx_last[None, :] + offs).reshape(B * N)
        matched = sc_gather(table, idx_all).reshape(B, N, 128)[..., :3]
        R, t = _ptransform_svd(temppc, matched)
        temppc_new = jnp.einsum('bij,bnj->bni', R, temppc) + t[..., None, :]
        temppc = jnp.where(converged, temppc, temppc_new)
        err = jnp.where(converged, err, errnew)
        have_err = jnp.logical_or(have_err, jnp.logical_not(converged))
        return it, temppc, err, have_err, converged

    init = (jnp.int32(0), p1, jnp.zeros((B,), dtype=p1.dtype),
            jnp.bool_(False), jnp.bool_(False))
    _, temppc, _, _, _ = lax.while_loop(cond_fn, body_fn, init)

    R, t = _ptransform_quat(p1, temppc)
    return jnp.concatenate([R, t[..., None]], axis=-1)


# speedup vs baseline: 1.0399x; 1.0399x over previous
import functools

import jax
import jax.numpy as jnp
from jax import lax
from jax.experimental import pallas as pl
from jax.experimental.pallas import tpu as pltpu
from jax.experimental.pallas import tpu_sc as plsc

_STEPLIM = 5
_TOL = 1e-4
_NB = 1024


def _knn_body(a_ref, p2t_ref, dist_ref, idx_ref):
    b = pl.program_id(0)
    nb_total = pl.num_programs(0)
    a = a_ref[0]
    p2t = p2t_ref[0]

    m = p2t.shape[1]
    a2 = jnp.sum(a * a, axis=1, keepdims=True)
    b2 = (p2t[0:1, :] * p2t[0:1, :] + p2t[1:2, :] * p2t[1:2, :]
          + p2t[2:3, :] * p2t[2:3, :])
    cross = lax.dot_general(a.astype(jnp.bfloat16), p2t.astype(jnp.bfloat16),
                            (((1,), (0,)), ((), ())),
                            preferred_element_type=jnp.float32)
    d2 = a2 + b2 - 2.0 * cross
    rowmin = jnp.maximum(jnp.min(d2, axis=1, keepdims=True), 0.0)
    dist_ref[...] = jnp.sqrt(rowmin)[None]

    @pl.when(b == nb_total - 1)
    def _():
        iota = lax.broadcasted_iota(jnp.int32, d2.shape, 1)
        cand = jnp.where(d2 <= rowmin, iota, m)
        idx_ref[...] = jnp.min(cand, axis=1, keepdims=True)[None]


def _make_knn_call(B, N, M, nb):
    nblk = N // nb
    return pl.pallas_call(
        _knn_body,
        grid=(B, nblk),
        in_specs=[
            pl.BlockSpec((1, nb, 3), lambda b, j: (b, j, 0)),
            pl.BlockSpec((1, 3, M), lambda b, j: (b, 0, 0)),
        ],
        out_specs=[
            pl.BlockSpec((1, nb, 1), lambda b, j: (b * nblk + j, 0, 0)),
            pl.BlockSpec((1, nb, 1), lambda b, j: (j, 0, 0)),
        ],
        out_shape=[
            jax.ShapeDtypeStruct((B * nblk, nb, 1), jnp.float32),
            jax.ShapeDtypeStruct((nblk, nb, 1), jnp.int32),
        ],
        compiler_params=pltpu.CompilerParams(
            dimension_semantics=("arbitrary", "arbitrary")),
    )


def _make_sc_gather(V, D, BB):
    info = plsc.get_sparse_core_info()
    nc, ns = info.num_cores, info.num_subcores
    nw = nc * ns
    b_per_w = BB // nw
    mesh = plsc.VectorSubcoreMesh(core_axis_name="c", subcore_axis_name="s")

    @functools.partial(
        pl.kernel, mesh=mesh,
        out_type=jax.ShapeDtypeStruct((BB, D), jnp.float32),
        scratch_types=[
            pltpu.VMEM((b_per_w,), jnp.int32),
            pltpu.VMEM((b_per_w, D), jnp.float32),
            pltpu.SemaphoreType.DMA,
        ],
    )
    def gather_k(table_hbm, idx_hbm, out_hbm, idx_v, rows_v, sem):
        wid = lax.axis_index("s") * nc + lax.axis_index("c")
        base = wid * b_per_w
        pltpu.sync_copy(idx_hbm.at[pl.ds(base, b_per_w)], idx_v)
        pltpu.async_copy(table_hbm.at[idx_v], rows_v, sem).wait()
        pltpu.sync_copy(rows_v, out_hbm.at[pl.ds(base, b_per_w)])

    return gather_k


def _quat_rot(H, sweeps=7):
    Sxx, Sxy, Sxz = H[..., 0, 0], H[..., 0, 1], H[..., 0, 2]
    Syx, Syy, Syz = H[..., 1, 0], H[..., 1, 1], H[..., 1, 2]
    Szx, Szy, Szz = H[..., 2, 0], H[..., 2, 1], H[..., 2, 2]
    A = jnp.stack([
        jnp.stack([Sxx + Syy + Szz, Syz - Szy, Szx - Sxz, Sxy - Syx], -1),
        jnp.stack([Syz - Szy, Sxx - Syy - Szz, Sxy + Syx, Szx + Sxz], -1),
        jnp.stack([Szx - Sxz, Sxy + Syx, -Sxx + Syy - Szz, Syz + Szy], -1),
        jnp.stack([Sxy - Syx, Szx + Sxz, Syz + Szy, -Sxx - Syy + Szz], -1),
    ], -2)
    Q = jnp.broadcast_to(jnp.eye(4, dtype=H.dtype), A.shape)
    for _ in range(sweeps):
        for (p, q) in ((0, 1), (0, 2), (0, 3), (1, 2), (1, 3), (2, 3)):
            x = A[..., p, p] - A[..., q, q]
            y = 2.0 * A[..., p, q]
            h = jnp.sqrt(x * x + y * y)
            ynz = y != 0.0
            t = jnp.where(x >= 0.0,
                          y / jnp.where(h + x > 0.0, h + x, 1.0),
                          jnp.where(ynz, (h - x) / jnp.where(ynz, y, 1.0),
                                    0.0))
            c = (1.0 / jnp.sqrt(1.0 + t * t))[..., None]
            s = (t[..., None]) * c
            Ap, Aq = A[..., p, :], A[..., q, :]
            A = A.at[..., p, :].set(c * Ap + s * Aq)
            A = A.at[..., q, :].set(-s * Ap + c * Aq)
            Ap, Aq = A[..., :, p], A[..., :, q]
            A = A.at[..., :, p].set(c * Ap + s * Aq)
            A = A.at[..., :, q].set(-s * Ap + c * Aq)
            Qp, Qq = Q[..., :, p], Q[..., :, q]
            Q = Q.at[..., :, p].set(c * Qp + s * Qq)
            Q = Q.at[..., :, q].set(-s * Qp + c * Qq)
    evals = jnp.diagonal(A, axis1=-2, axis2=-1)
    k = jnp.argmax(evals, axis=-1)
    qv = jnp.take_along_axis(
        Q, jnp.broadcast_to(k[..., None, None], Q.shape[:-1] + (1,)), axis=-1
    )[..., 0]
    qv = qv / jnp.sqrt(jnp.sum(qv * qv, axis=-1, keepdims=True))
    w, x, y, z = qv[..., 0], qv[..., 1], qv[..., 2], qv[..., 3]
    return jnp.stack([
        jnp.stack([1 - 2 * (y * y + z * z), 2 * (x * y - w * z),
                   2 * (x * z + w * y)], -1),
        jnp.stack([2 * (x * y + w * z), 1 - 2 * (x * x + z * z),
                   2 * (y * z - w * x)], -1),
        jnp.stack([2 * (x * z - w * y), 2 * (y * z + w * x),
                   1 - 2 * (x * x + y * y)], -1),
    ], -2)


def _ptransform_svd(pa, pb):
    c1 = jnp.mean(pa, axis=-2, keepdims=True)
    c2 = jnp.mean(pb, axis=-2, keepdims=True)
    H = jnp.einsum('bni,bnj->bij', pa - c1, pb - c2)
    U, S, Vt = jnp.linalg.svd(H)
    V = jnp.swapaxes(Vt, -1, -2)
    Ut = jnp.swapaxes(U, -1, -2)
    d = jnp.linalg.det(jnp.matmul(V, Ut))
    s = jnp.where(d < 0, -1.0, 1.0)
    D = jnp.stack([jnp.ones_like(s), jnp.ones_like(s), s], axis=-1)
    R = jnp.matmul(V * D[..., None, :], Ut)
    t = c2[..., 0, :] - jnp.einsum('bij,bj->bi', R, c1[..., 0, :])
    return R, t


def _ptransform_quat(pa, pb):
    c1 = jnp.mean(pa, axis=-2, keepdims=True)
    c2 = jnp.mean(pb, axis=-2, keepdims=True)
    H = jnp.einsum('bni,bnj->bij', pa - c1, pb - c2)
    R = _quat_rot(H)
    t = c2[..., 0, :] - jnp.einsum('bij,bj->bi', R, c1[..., 0, :])
    return R, t


def kernel(p1, p2):
    B, N, _ = p1.shape
    M = p2.shape[1]
    p2t = jnp.swapaxes(p2, 1, 2)
    table = jnp.pad(p2, ((0, 0), (0, 0), (0, 125))).reshape(B * M, 128)
    offs = (jnp.arange(B, dtype=jnp.int32) * M)[:, None]

    knn_call = _make_knn_call(B, N, M, _NB)
    sc_gather = _make_sc_gather(B * M, 128, B * N)
    nblk = N // _NB

    def cond_fn(carry):
        it, temppc, err, have_err, done = carry
        return (it <= _STEPLIM) & jnp.logical_not(done)

    def body_fn(carry):
        it, temppc, err, have_err, done = carry
        it = it + 1
        dist_o, idx_o = knn_call(temppc, p2t)
        knndist = dist_o.reshape(B, N)
        idx_last = idx_o.reshape(N)
        errnew = jnp.mean(knndist, axis=-1)
        converged = have_err & jnp.all(jnp.abs((errnew - err) / err) < _TOL)
        idx_all = (idx_last[None, :] + offs).reshape(B * N)
        matched = sc_gather(table, idx_all).reshape(B, N, 128)[..., :3]
        R, t = _ptransform_svd(temppc, matched)
        temppc_new = jnp.einsum('bij,bnj->bni', R, temppc) + t[..., None, :]
        temppc = jnp.where(converged, temppc, temppc_new)
        err = jnp.where(converged, err, errnew)
        have_err = jnp.logical_or(have_err, jnp.logical_not(converged))
        return it, temppc, err, have_err, converged

    init = (jnp.int32(0), p1, jnp.zeros((B,), dtype=p1.dtype),
            jnp.bool_(False), jnp.bool_(False))
    _, temppc, _, _, _ = lax.while_loop(cond_fn, body_fn, init)

    R, t = _ptransform_svd(p1, temppc)
    return jnp.concatenate([R, t[..., None]], axis=-1)


# speedup vs baseline: 1.0500x; 1.0098x over previous
import functools

import jax
import jax.numpy as jnp
from jax import lax
from jax.experimental import pallas as pl
from jax.experimental.pallas import tpu as pltpu
from jax.experimental.pallas import tpu_sc as plsc

_STEPLIM = 5
_TOL = 1e-4
_NB = 2048


def _knn_body(a_ref, p2t_ref, dist_ref, idx_ref):
    b = pl.program_id(0)
    nb_total = pl.num_programs(0)
    a = a_ref[0]
    p2t = p2t_ref[0]

    m = p2t.shape[1]
    a2 = jnp.sum(a * a, axis=1, keepdims=True)
    b2 = (p2t[0:1, :] * p2t[0:1, :] + p2t[1:2, :] * p2t[1:2, :]
          + p2t[2:3, :] * p2t[2:3, :])
    cross = lax.dot_general(a.astype(jnp.bfloat16), p2t.astype(jnp.bfloat16),
                            (((1,), (0,)), ((), ())),
                            preferred_element_type=jnp.float32)
    d2 = a2 + b2 - 2.0 * cross
    rowmin = jnp.maximum(jnp.min(d2, axis=1, keepdims=True), 0.0)
    dist_ref[...] = jnp.sqrt(rowmin)[None]

    @pl.when(b == nb_total - 1)
    def _():
        iota = lax.broadcasted_iota(jnp.int32, d2.shape, 1)
        cand = jnp.where(d2 <= rowmin, iota, m)
        idx_ref[...] = jnp.min(cand, axis=1, keepdims=True)[None]


def _make_knn_call(B, N, M, nb):
    nblk = N // nb
    return pl.pallas_call(
        _knn_body,
        grid=(B, nblk),
        in_specs=[
            pl.BlockSpec((1, nb, 3), lambda b, j: (b, j, 0)),
            pl.BlockSpec((1, 3, M), lambda b, j: (b, 0, 0)),
        ],
        out_specs=[
            pl.BlockSpec((1, nb, 1), lambda b, j: (b * nblk + j, 0, 0)),
            pl.BlockSpec((1, nb, 1), lambda b, j: (j, 0, 0)),
        ],
        out_shape=[
            jax.ShapeDtypeStruct((B * nblk, nb, 1), jnp.float32),
            jax.ShapeDtypeStruct((nblk, nb, 1), jnp.int32),
        ],
        compiler_params=pltpu.CompilerParams(
            dimension_semantics=("arbitrary", "arbitrary")),
    )


def _make_sc_gather(V, D, BB):
    info = plsc.get_sparse_core_info()
    nc, ns = info.num_cores, info.num_subcores
    nw = nc * ns
    b_per_w = BB // nw
    mesh = plsc.VectorSubcoreMesh(core_axis_name="c", subcore_axis_name="s")

    @functools.partial(
        pl.kernel, mesh=mesh,
        out_type=jax.ShapeDtypeStruct((BB, D), jnp.float32),
        scratch_types=[
            pltpu.VMEM((b_per_w,), jnp.int32),
            pltpu.VMEM((b_per_w, D), jnp.float32),
            pltpu.SemaphoreType.DMA,
        ],
    )
    def gather_k(table_hbm, idx_hbm, out_hbm, idx_v, rows_v, sem):
        wid = lax.axis_index("s") * nc + lax.axis_index("c")
        base = wid * b_per_w
        pltpu.sync_copy(idx_hbm.at[pl.ds(base, b_per_w)], idx_v)
        pltpu.async_copy(table_hbm.at[idx_v], rows_v, sem).wait()
        pltpu.sync_copy(rows_v, out_hbm.at[pl.ds(base, b_per_w)])

    return gather_k


def _quat_rot(H, sweeps=7):
    Sxx, Sxy, Sxz = H[..., 0, 0], H[..., 0, 1], H[..., 0, 2]
    Syx, Syy, Syz = H[..., 1, 0], H[..., 1, 1], H[..., 1, 2]
    Szx, Szy, Szz = H[..., 2, 0], H[..., 2, 1], H[..., 2, 2]
    A = jnp.stack([
        jnp.stack([Sxx + Syy + Szz, Syz - Szy, Szx - Sxz, Sxy - Syx], -1),
        jnp.stack([Syz - Szy, Sxx - Syy - Szz, Sxy + Syx, Szx + Sxz], -1),
        jnp.stack([Szx - Sxz, Sxy + Syx, -Sxx + Syy - Szz, Syz + Szy], -1),
        jnp.stack([Sxy - Syx, Szx + Sxz, Syz + Szy, -Sxx - Syy + Szz], -1),
    ], -2)
    Q = jnp.broadcast_to(jnp.eye(4, dtype=H.dtype), A.shape)
    for _ in range(sweeps):
        for (p, q) in ((0, 1), (0, 2), (0, 3), (1, 2), (1, 3), (2, 3)):
            x = A[..., p, p] - A[..., q, q]
            y = 2.0 * A[..., p, q]
            h = jnp.sqrt(x * x + y * y)
            ynz = y != 0.0
            t = jnp.where(x >= 0.0,
                          y / jnp.where(h + x > 0.0, h + x, 1.0),
                          jnp.where(ynz, (h - x) / jnp.where(ynz, y, 1.0),
                                    0.0))
            c = (1.0 / jnp.sqrt(1.0 + t * t))[..., None]
            s = (t[..., None]) * c
            Ap, Aq = A[..., p, :], A[..., q, :]
            A = A.at[..., p, :].set(c * Ap + s * Aq)
            A = A.at[..., q, :].set(-s * Ap + c * Aq)
            Ap, Aq = A[..., :, p], A[..., :, q]
            A = A.at[..., :, p].set(c * Ap + s * Aq)
            A = A.at[..., :, q].set(-s * Ap + c * Aq)
            Qp, Qq = Q[..., :, p], Q[..., :, q]
            Q = Q.at[..., :, p].set(c * Qp + s * Qq)
            Q = Q.at[..., :, q].set(-s * Qp + c * Qq)
    evals = jnp.diagonal(A, axis1=-2, axis2=-1)
    k = jnp.argmax(evals, axis=-1)
    qv = jnp.take_along_axis(
        Q, jnp.broadcast_to(k[..., None, None], Q.shape[:-1] + (1,)), axis=-1
    )[..., 0]
    qv = qv / jnp.sqrt(jnp.sum(qv * qv, axis=-1, keepdims=True))
    w, x, y, z = qv[..., 0], qv[..., 1], qv[..., 2], qv[..., 3]
    return jnp.stack([
        jnp.stack([1 - 2 * (y * y + z * z), 2 * (x * y - w * z),
                   2 * (x * z + w * y)], -1),
        jnp.stack([2 * (x * y + w * z), 1 - 2 * (x * x + z * z),
                   2 * (y * z - w * x)], -1),
        jnp.stack([2 * (x * z - w * y), 2 * (y * z + w * x),
                   1 - 2 * (x * x + y * y)], -1),
    ], -2)


def _ptransform_svd(pa, pb):
    c1 = jnp.mean(pa, axis=-2, keepdims=True)
    c2 = jnp.mean(pb, axis=-2, keepdims=True)
    H = jnp.einsum('bni,bnj->bij', pa - c1, pb - c2)
    U, S, Vt = jnp.linalg.svd(H)
    V = jnp.swapaxes(Vt, -1, -2)
    Ut = jnp.swapaxes(U, -1, -2)
    d = jnp.linalg.det(jnp.matmul(V, Ut))
    s = jnp.where(d < 0, -1.0, 1.0)
    D = jnp.stack([jnp.ones_like(s), jnp.ones_like(s), s], axis=-1)
    R = jnp.matmul(V * D[..., None, :], Ut)
    t = c2[..., 0, :] - jnp.einsum('bij,bj->bi', R, c1[..., 0, :])
    return R, t


def _ptransform_quat(pa, pb):
    c1 = jnp.mean(pa, axis=-2, keepdims=True)
    c2 = jnp.mean(pb, axis=-2, keepdims=True)
    H = jnp.einsum('bni,bnj->bij', pa - c1, pb - c2)
    R = _quat_rot(H)
    t = c2[..., 0, :] - jnp.einsum('bij,bj->bi', R, c1[..., 0, :])
    return R, t


def kernel(p1, p2):
    B, N, _ = p1.shape
    M = p2.shape[1]
    p2t = jnp.swapaxes(p2, 1, 2)
    table = jnp.pad(p2, ((0, 0), (0, 0), (0, 125))).reshape(B * M, 128)
    offs = (jnp.arange(B, dtype=jnp.int32) * M)[:, None]

    knn_call = _make_knn_call(B, N, M, _NB)
    sc_gather = _make_sc_gather(B * M, 128, B * N)
    nblk = N // _NB

    def cond_fn(carry):
        it, temppc, err, have_err, done = carry
        return (it <= _STEPLIM) & jnp.logical_not(done)

    def body_fn(carry):
        it, temppc, err, have_err, done = carry
        it = it + 1
        dist_o, idx_o = knn_call(temppc, p2t)
        knndist = dist_o.reshape(B, N)
        idx_last = idx_o.reshape(N)
        errnew = jnp.mean(knndist, axis=-1)
        converged = have_err & jnp.all(jnp.abs((errnew - err) / err) < _TOL)
        idx_all = (idx_last[None, :] + offs).reshape(B * N)
        matched = sc_gather(table, idx_all).reshape(B, N, 128)[..., :3]
        R, t = _ptransform_svd(temppc, matched)
        temppc_new = jnp.einsum('bij,bnj->bni', R, temppc) + t[..., None, :]
        temppc = jnp.where(converged, temppc, temppc_new)
        err = jnp.where(converged, err, errnew)
        have_err = jnp.logical_or(have_err, jnp.logical_not(converged))
        return it, temppc, err, have_err, converged

    init = (jnp.int32(0), p1, jnp.zeros((B,), dtype=p1.dtype),
            jnp.bool_(False), jnp.bool_(False))
    _, temppc, _, _, _ = lax.while_loop(cond_fn, body_fn, init)

    R, t = _ptransform_svd(p1, temppc)
    return jnp.concatenate([R, t[..., None]], axis=-1)
